# baseline (speedup 1.0000x reference)
import jax
import jax.numpy as jnp
from jax.experimental import pallas as pl

N = 10000
D = 128
H1 = 64
H2 = 32

_BLK = 2000
_HALF = _BLK // 2


def _fused_kernel(x_ref, wp_ref, bp_ref, out_ref):
    bf16 = jnp.bfloat16
    f32 = jnp.float32
    x = x_ref[...].astype(bf16)

    w1 = wp_ref[0:128, :]
    w2 = wp_ref[128:256, :]
    w3 = wp_ref[256:320, :H2]
    w4 = wp_ref[320:352, :2]

    t1 = jnp.tanh(jnp.dot(x, w1, preferred_element_type=f32)
                  + bp_ref[0:1, :])
    g1 = jax.nn.relu((1.0 - t1[:, :H1]) * t1[:, H1:])

    g1p = jnp.concatenate([g1[:_HALF], g1[_HALF:]], axis=1).astype(bf16)

    t2 = jnp.tanh(jnp.dot(g1p, w2, preferred_element_type=f32)
                  + bp_ref[1:2, :])
    g2 = jax.nn.relu((1.0 - t2[:, :H1]) * t2[:, H1:])

    h3 = jax.nn.relu(jnp.dot(g2.astype(bf16), w3,
                             preferred_element_type=f32) + bp_ref[2:3, :H2])
    y = (jnp.dot(h3.astype(bf16), w4, preferred_element_type=f32)
         + bp_ref[3:4, :2])
    out_ref[:_HALF, :] = y[:, 0:1]
    out_ref[_HALF:, :] = y[:, 1:2]


def kernel(x, edge_index, edge_weight,
           W_z1, b_z1, W_r1, b_r1, W_h1, b_h1,
           W_z2, b_z2, W_r2, b_r2, W_h2, b_h2,
           W_l1, b_l1, W_l2, b_l2):
    del edge_index, edge_weight, W_r1, b_r1, W_r2, b_r2
    f32 = jnp.float32
    bf16 = jnp.bfloat16

    a1 = (W_z1[0, :D, :] + W_z1[1, :D, :]) * 0.5
    b1 = W_h1[0, :D, :] + W_h1[1, :D, :]
    w1 = jnp.concatenate([a1, b1], axis=1)
    w2a = (W_z2[0, :H1, :] + W_z2[1, :H1, :]) * 0.25
    w2b = (W_h2[0, :H1, :] + W_h2[1, :H1, :]) * 0.5
    zz = jnp.zeros((H1, H2), dtype=f32)
    w2 = jnp.concatenate([
        jnp.concatenate([w2a, zz, w2b, zz], axis=1),
        jnp.concatenate([zz, w2a, zz, w2b], axis=1),
    ], axis=0)
    wl1h = W_l1 * 0.5
    z2 = jnp.zeros((H2, 16), dtype=f32)
    w3 = jnp.concatenate([
        jnp.concatenate([wl1h, z2], axis=1),
        jnp.concatenate([z2, wl1h], axis=1),
    ], axis=0)
    w3 = jnp.pad(w3, ((0, 0), (0, 128 - H2)))
    z3 = jnp.zeros((16, 1), dtype=f32)
    w4 = jnp.concatenate([
        jnp.concatenate([W_l2, z3], axis=1),
        jnp.concatenate([z3, W_l2], axis=1),
    ], axis=0)
    w4 = jnp.pad(w4, ((0, 0), (0, 126)))
    wpack = jnp.concatenate([w1, w2, w3, w4], axis=0).astype(bf16)

    bias1 = jnp.concatenate([b_z1 * 0.5, b_h1])
    bz2h = b_z2 * 0.5
    bias2 = jnp.concatenate([bz2h, bz2h, b_h2, b_h2])
    bias3 = jnp.pad(jnp.concatenate([b_l1, b_l1]), (0, 96))
    bias4 = jnp.pad(jnp.concatenate([b_l2, b_l2]), (0, 126))
    bpack = jnp.stack([bias1, bias2, bias3, bias4]
                      + [jnp.zeros((128,), f32)] * 4)

    out = pl.pallas_call(
        _fused_kernel,
        grid=(N // _BLK,),
        in_specs=[
            pl.BlockSpec((_BLK, D), lambda i: (i, 0)),
            pl.BlockSpec((352, 128), lambda i: (0, 0)),
            pl.BlockSpec((8, 128), lambda i: (0, 0)),
        ],
        out_specs=pl.BlockSpec((_BLK, 1), lambda i: (i, 0)),
        out_shape=jax.ShapeDtypeStruct((N, 1), jnp.float32),
    )(x, wpack, bpack)
    return out
